# baseline (speedup 1.0000x reference)
import jax
import jax.numpy as jnp
from jax.experimental import pallas as pl
from jax.experimental.pallas import tpu as pltpu

_N_BLK = 1024


def _mm_kernel(it_ref, ut_ref, o_ref):
    o_ref[...] = jax.lax.dot_general(
        it_ref[...],
        ut_ref[...],
        dimension_numbers=(((0,), (0,)), ((), ())),
        preferred_element_type=jnp.float32,
    )


def kernel(u_g_embeddings, i_g_embeddings):
    M, K = u_g_embeddings.shape
    N = i_g_embeddings.shape[0]
    ut = u_g_embeddings.T
    it = i_g_embeddings.T
    out_t = pl.pallas_call(
        _mm_kernel,
        grid=(pl.cdiv(N, _N_BLK),),
        in_specs=[
            pl.BlockSpec((K, _N_BLK), lambda j: (0, j)),
            pl.BlockSpec((K, M), lambda j: (0, 0)),
        ],
        out_specs=pl.BlockSpec((_N_BLK, M), lambda j: (j, 0)),
        out_shape=jax.ShapeDtypeStruct((N, M), jnp.float32),
        compiler_params=pltpu.CompilerParams(
            dimension_semantics=("parallel",),
        ),
    )(it, ut)
    return out_t.T


# speedup vs baseline: 1.1093x; 1.1093x over previous
import jax
import jax.numpy as jnp
from jax.experimental import pallas as pl
from jax.experimental.pallas import tpu as pltpu

_N_BLK = 2048


def _mm_kernel(it_ref, ut_ref, o_ref):
    o_ref[...] = jax.lax.dot_general(
        it_ref[...],
        ut_ref[...],
        dimension_numbers=(((0,), (0,)), ((), ())),
        preferred_element_type=jnp.float32,
    )


def kernel(u_g_embeddings, i_g_embeddings):
    M, K = u_g_embeddings.shape
    N = i_g_embeddings.shape[0]
    ut = u_g_embeddings.T
    it = i_g_embeddings.T
    out_t = pl.pallas_call(
        _mm_kernel,
        grid=(pl.cdiv(N, _N_BLK),),
        in_specs=[
            pl.BlockSpec((K, _N_BLK), lambda j: (0, j)),
            pl.BlockSpec((K, M), lambda j: (0, 0)),
        ],
        out_specs=pl.BlockSpec((_N_BLK, M), lambda j: (j, 0)),
        out_shape=jax.ShapeDtypeStruct((N, M), jnp.float32),
        compiler_params=pltpu.CompilerParams(
            dimension_semantics=("parallel",),
        ),
    )(it, ut)
    return out_t.T
